# baseline (speedup 1.0000x reference)
import jax
import jax.numpy as jnp
from jax.experimental import pallas as pl


_ROWS, _COLS = 16384, 128
_BLOCK_ROWS = 16384
_GRID = _ROWS // _BLOCK_ROWS


def _copy_set_kernel(x_ref, o_ref):
    o_ref[...] = x_ref[...]

    @pl.when(pl.program_id(0) == 0)
    def _():
        head = x_ref[pl.ds(0, 8), :]
        rows = jax.lax.broadcasted_iota(jnp.int32, (8, _COLS), 0)
        cols = jax.lax.broadcasted_iota(jnp.int32, (8, _COLS), 1)
        hit = (rows == 0) & (cols == 0)
        o_ref[pl.ds(0, 8), :] = jnp.where(hit, jnp.float32(3.0), head)


def kernel(x):
    return pl.pallas_call(
        _copy_set_kernel,
        grid=(_GRID,),
        in_specs=[pl.BlockSpec((_BLOCK_ROWS, _COLS), lambda i: (i, 0))],
        out_specs=pl.BlockSpec((_BLOCK_ROWS, _COLS), lambda i: (i, 0)),
        out_shape=jax.ShapeDtypeStruct((_ROWS, _COLS), x.dtype),
    )(x)


# speedup vs baseline: 1.1740x; 1.1740x over previous
import jax
import jax.numpy as jnp
from jax.experimental import pallas as pl
from jax.experimental.pallas import tpu as pltpu


_ROWS, _COLS = 16384, 128
_NCHUNKS = 16
_CHUNK = _ROWS // _NCHUNKS


def _copy_kernel(x_hbm, o_hbm, buf, sem_in, sem_out):
    ins = []
    for i in range(_NCHUNKS):
        cp = pltpu.make_async_copy(
            x_hbm.at[pl.ds(i * _CHUNK, _CHUNK), :],
            buf.at[pl.ds(i * _CHUNK, _CHUNK), :],
            sem_in.at[i],
        )
        cp.start()
        ins.append(cp)

    outs = []
    for i in range(_NCHUNKS):
        ins[i].wait()
        if i == 0:
            lane = jax.lax.iota(jnp.int32, _COLS)
            head = buf[0, :]
            buf[0, :] = jnp.where(lane == 0, jnp.float32(3.0), head)
        cp = pltpu.make_async_copy(
            buf.at[pl.ds(i * _CHUNK, _CHUNK), :],
            o_hbm.at[pl.ds(i * _CHUNK, _CHUNK), :],
            sem_out.at[i],
        )
        cp.start()
        outs.append(cp)

    for cp in outs:
        cp.wait()


def kernel(x):
    return pl.pallas_call(
        _copy_kernel,
        in_specs=[pl.BlockSpec(memory_space=pl.ANY)],
        out_specs=pl.BlockSpec(memory_space=pl.ANY),
        out_shape=jax.ShapeDtypeStruct((_ROWS, _COLS), x.dtype),
        scratch_shapes=[
            pltpu.VMEM((_ROWS, _COLS), jnp.float32),
            pltpu.SemaphoreType.DMA((_NCHUNKS,)),
            pltpu.SemaphoreType.DMA((_NCHUNKS,)),
        ],
    )(x)


# speedup vs baseline: 1.2405x; 1.0566x over previous
import jax
import jax.numpy as jnp
from jax.experimental import pallas as pl
from jax.experimental.pallas import tpu as pltpu


_ROWS, _COLS = 16384, 128
_NCHUNKS = 4
_CHUNK = _ROWS // _NCHUNKS


def _copy_kernel(x_hbm, o_hbm, buf, sem_in, sem_out):
    ins = []
    for i in range(_NCHUNKS):
        cp = pltpu.make_async_copy(
            x_hbm.at[pl.ds(i * _CHUNK, _CHUNK), :],
            buf.at[pl.ds(i * _CHUNK, _CHUNK), :],
            sem_in.at[i],
        )
        cp.start()
        ins.append(cp)

    outs = []
    for i in range(_NCHUNKS):
        ins[i].wait()
        if i == 0:
            lane = jax.lax.iota(jnp.int32, _COLS)
            head = buf[0, :]
            buf[0, :] = jnp.where(lane == 0, jnp.float32(3.0), head)
        cp = pltpu.make_async_copy(
            buf.at[pl.ds(i * _CHUNK, _CHUNK), :],
            o_hbm.at[pl.ds(i * _CHUNK, _CHUNK), :],
            sem_out.at[i],
        )
        cp.start()
        outs.append(cp)

    for cp in outs:
        cp.wait()


def kernel(x):
    return pl.pallas_call(
        _copy_kernel,
        in_specs=[pl.BlockSpec(memory_space=pl.ANY)],
        out_specs=pl.BlockSpec(memory_space=pl.ANY),
        out_shape=jax.ShapeDtypeStruct((_ROWS, _COLS), x.dtype),
        scratch_shapes=[
            pltpu.VMEM((_ROWS, _COLS), jnp.float32),
            pltpu.SemaphoreType.DMA((_NCHUNKS,)),
            pltpu.SemaphoreType.DMA((_NCHUNKS,)),
        ],
    )(x)


# speedup vs baseline: 1.2826x; 1.0339x over previous
import jax
import jax.numpy as jnp
from jax.experimental import pallas as pl
from jax.experimental.pallas import tpu as pltpu


_ROWS, _COLS = 16384, 128
_CHUNKS = (512, 1536, 3584, 4096, 3584, 1536, 1024, 512)
assert sum(_CHUNKS) == _ROWS
_OFFS = tuple(sum(_CHUNKS[:i]) for i in range(len(_CHUNKS)))
_N = len(_CHUNKS)


def _copy_kernel(x_hbm, o_hbm, buf, sem_in, sem_out):
    ins = []
    for i in range(_N):
        cp = pltpu.make_async_copy(
            x_hbm.at[pl.ds(_OFFS[i], _CHUNKS[i]), :],
            buf.at[pl.ds(_OFFS[i], _CHUNKS[i]), :],
            sem_in.at[i],
        )
        cp.start()
        ins.append(cp)

    outs = []
    for i in range(_N):
        ins[i].wait()
        if i == 0:
            lane = jax.lax.iota(jnp.int32, _COLS)
            head = buf[0, :]
            buf[0, :] = jnp.where(lane == 0, jnp.float32(3.0), head)
        cp = pltpu.make_async_copy(
            buf.at[pl.ds(_OFFS[i], _CHUNKS[i]), :],
            o_hbm.at[pl.ds(_OFFS[i], _CHUNKS[i]), :],
            sem_out.at[i],
        )
        cp.start()
        outs.append(cp)

    for cp in outs:
        cp.wait()


def kernel(x):
    return pl.pallas_call(
        _copy_kernel,
        in_specs=[pl.BlockSpec(memory_space=pl.ANY)],
        out_specs=pl.BlockSpec(memory_space=pl.ANY),
        out_shape=jax.ShapeDtypeStruct((_ROWS, _COLS), x.dtype),
        scratch_shapes=[
            pltpu.VMEM((_ROWS, _COLS), jnp.float32),
            pltpu.SemaphoreType.DMA((_N,)),
            pltpu.SemaphoreType.DMA((_N,)),
        ],
    )(x)
